# baseline (speedup 1.0000x reference)
import numpy as np
import jax
import jax.numpy as jnp
from jax.experimental import pallas as pl
from jax.experimental.pallas import tpu as pltpu

SEQ = 192
PRED = 96
DIN = 8
DEMB = 16
KS = 25
HID = 64
G = 8
END = DEMB + 2
B = 8
N = 512
T = SEQ + PRED


def _avg_matrix():
    pad = (KS - 1) // 2
    A = np.zeros((SEQ, SEQ), np.float32)
    for t in range(SEQ):
        for u in range(t - pad, t + pad + 1):
            j = min(max(u, 0), SEQ - 1)
            A[j, t] += 1.0 / KS
    return jnp.asarray(A)


def _stats_kernel(gi_ref, x_ref, A_ref, Ws_ref, bs_ref, Wt_ref, bt_ref,
                  Wr_ref, br_ref,
                  gin_ref, gout_ref, sf_ref, tf_ref, sas_ref, tas_ref):
    gv = gi_ref[...]
    oh = (jax.lax.broadcasted_iota(jnp.int32, (G, N), 0) == gv
          ).astype(jnp.float32)
    cnt = jnp.sum(oh, axis=1, keepdims=True)
    inv = 1.0 / jnp.maximum(cnt, 1.0)
    A = A_ref[...]
    gs = jnp.stack(
        [jnp.dot(oh, x_ref[b], preferred_element_type=jnp.float32)
         for b in range(B)], axis=1)
    gin = gs * inv[:, :, None]
    gin_ref[...] = gin
    gout_ref[...] = jnp.stack(
        [jnp.dot(gin[g], Wr_ref[g].T, preferred_element_type=jnp.float32)
         + br_ref[g] for g in range(G)], axis=0)
    gtr = jnp.dot(gs.reshape(G * B, SEQ), A,
                  preferred_element_type=jnp.float32).reshape(G, B, SEQ)
    gse = gs - gtr
    totx = jnp.sum(gs, axis=0)
    tf = jnp.dot(totx, A, preferred_element_type=jnp.float32)
    tf_ref[...] = tf
    sf_ref[...] = totx - tf
    sas = jnp.zeros((B, PRED), jnp.float32)
    tas = jnp.zeros((B, PRED), jnp.float32)
    for g in range(G):
        sas = sas + jnp.dot(gse[g], Ws_ref[g].T,
                            preferred_element_type=jnp.float32)
        tas = tas + jnp.dot(gtr[g], Wt_ref[g].T,
                            preferred_element_type=jnp.float32)
    sas_ref[...] = sas + jnp.sum(cnt * bs_ref[...], axis=0, keepdims=True)
    tas_ref[...] = tas + jnp.sum(cnt * bt_ref[...], axis=0, keepdims=True)


def _series_kernel(gi_sref, x_ref, dy_ref, A_ref, We_ref, be_ref,
                   Ws_ref, bs_ref, Wt_ref, bt_ref,
                   W1_ref, b1_ref, W2_ref, b2_ref, gin_ref, gout_ref,
                   of_ref, oa_ref, dye_ref):
    step = pl.program_id(0)
    A = A_ref[...]
    WeT = We_ref[...].T
    be = be_ref[...]
    g = gi_sref[step]
    xb = x_ref[0]
    tr = jnp.dot(xb, A, preferred_element_type=jnp.float32)
    se = xb - tr
    dmat = dy_ref[:, 0, :, :]
    dye3 = jnp.maximum(
        jnp.dot(dmat.reshape(B * T, DIN), WeT,
                preferred_element_type=jnp.float32) + be,
        0.0).reshape(B, T, DEMB)
    dye_acc = dye3
    Wsg = Ws_ref[g]
    Wtg = Wt_ref[g]
    so = jnp.dot(se, Wsg.T, preferred_element_type=jnp.float32) + bs_ref[g]
    to = jnp.dot(tr, Wtg.T, preferred_element_type=jnp.float32) + bt_ref[g]
    W1g = W1_ref[g]
    w1s = W1g[:, 0][None, None, :]
    w1t = W1g[:, 1][None, None, :]
    W1dT = W1g[:, 2:].T
    b1g = b1_ref[g][None, :, :]
    w2g = W2_ref[g][None, :, :]
    b2g = b2_ref[g]
    dyf = dye3[:, :SEQ, :]
    h1f = jnp.maximum(
        jnp.dot(dyf.reshape(B * SEQ, DEMB), W1dT,
                preferred_element_type=jnp.float32).reshape(B, SEQ, HID)
        + se[:, :, None] * w1s + tr[:, :, None] * w1t + b1g, 0.0)
    of = jnp.sum(h1f * w2g, axis=-1) + b2g + gin_ref[g]
    of_ref[0] = of
    dya = dye3[:, SEQ:, :]
    h1a = jnp.maximum(
        jnp.dot(dya.reshape(B * PRED, DEMB), W1dT,
                preferred_element_type=jnp.float32).reshape(B, PRED, HID)
        + so[:, :, None] * w1s + to[:, :, None] * w1t + b1g, 0.0)
    oa = jnp.sum(h1a * w2g, axis=-1) + b2g + gout_ref[g]
    oa_ref[0] = oa

    @pl.when(step == 0)
    def _():
        dye_ref[...] = dye_acc

    @pl.when(step != 0)
    def _():
        dye_ref[...] = dye_ref[...] + dye_acc


def _gru_kernel(sf_ref, tf_ref, sas_ref, tas_ref, dye_ref,
                Wih_ref, Whh_ref, bih_ref, bhh_ref, Wa_ref, ba_ref,
                out_ref, g_ref, hs_ref):
    invN = 1.0 / N
    Wih = Wih_ref[...]
    wi_s = Wih[:, 0][None, None, :]
    wi_t = Wih[:, 1][None, None, :]
    WidT = Wih[:, 2:].T
    bih = bih_ref[...][None, :, :]
    dye = dye_ref[...] * invN
    H3 = 3 * HID
    Gf = (jnp.dot(dye[:, :SEQ, :].reshape(B * SEQ, DEMB), WidT,
                  preferred_element_type=jnp.float32).reshape(B, SEQ, H3)
          + (sf_ref[...] * invN)[:, :, None] * wi_s
          + (tf_ref[...] * invN)[:, :, None] * wi_t + bih)
    Ga = (jnp.dot(dye[:, SEQ:, :].reshape(B * PRED, DEMB), WidT,
                  preferred_element_type=jnp.float32).reshape(B, PRED, H3)
          + (sas_ref[...] * invN)[:, :, None] * wi_s
          + (tas_ref[...] * invN)[:, :, None] * wi_t + bih)
    Ga = jnp.concatenate([Ga, jnp.zeros((B, SEQ - PRED, H3), jnp.float32)],
                         axis=1)
    g_ref[...] = jnp.concatenate([Gf, Ga], axis=0)
    WhhT = Whh_ref[...].T
    bhh = bhh_ref[...]

    def body(t, h):
        git = g_ref[:, pl.ds(t, 1), :][:, 0, :]
        gh = jnp.dot(h, WhhT, preferred_element_type=jnp.float32) + bhh
        r = jax.nn.sigmoid(git[:, :HID] + gh[:, :HID])
        z = jax.nn.sigmoid(git[:, HID:2 * HID] + gh[:, HID:2 * HID])
        nc = jnp.tanh(git[:, 2 * HID:] + r * gh[:, 2 * HID:])
        hn = (1.0 - z) * nc + z * h
        hs_ref[:, pl.ds(t, 1), :] = hn[:, None, :]
        return hn

    h0 = jnp.zeros((2 * B, HID), jnp.float32)
    jax.lax.fori_loop(0, SEQ, body, h0)
    wa = Wa_ref[...][None, :, :]
    out_ref[...] = jax.nn.sigmoid(
        jnp.sum(hs_ref[...] * wa, axis=-1) + ba_ref[0, 0])


def kernel(x, dy, gi, We, be, Ws, bs, Wt, bt, Wr, br, W1, b1, W2, b2,
           Wa, ba, Wih, Whh, bih, bhh):
    f32 = jnp.float32
    A = _avg_matrix()
    gi2 = gi.reshape(1, N)
    gin, gout, sf, tf, sas, tas = pl.pallas_call(
        _stats_kernel,
        out_shape=[
            jax.ShapeDtypeStruct((G, B, SEQ), f32),
            jax.ShapeDtypeStruct((G, B, PRED), f32),
            jax.ShapeDtypeStruct((B, SEQ), f32),
            jax.ShapeDtypeStruct((B, SEQ), f32),
            jax.ShapeDtypeStruct((B, PRED), f32),
            jax.ShapeDtypeStruct((B, PRED), f32),
        ],
    )(gi2, x, A, Ws, bs, Wt, bt, Wr, br.reshape(G, 1, PRED))

    full = lambda shape: pl.BlockSpec(shape, lambda s, gref: (0,) * len(shape))
    grid_spec = pltpu.PrefetchScalarGridSpec(
        num_scalar_prefetch=1,
        grid=(N,),
        in_specs=[
            pl.BlockSpec((1, B, SEQ), lambda s, gref: (s, 0, 0)),
            pl.BlockSpec((B, 1, T, DIN), lambda s, gref: (0, s, 0, 0)),
            full((SEQ, SEQ)),
            full((DEMB, DIN)),
            full((1, DEMB)),
            full((G, PRED, SEQ)),
            full((G, 1, PRED)),
            full((G, PRED, SEQ)),
            full((G, 1, PRED)),
            full((G, HID, END)),
            full((G, 1, HID)),
            full((G, 1, HID)),
            full((G, 1, 1)),
            full((G, B, SEQ)),
            full((G, B, PRED)),
        ],
        out_specs=[
            pl.BlockSpec((1, B, SEQ), lambda s, gref: (s, 0, 0)),
            pl.BlockSpec((1, B, PRED), lambda s, gref: (s, 0, 0)),
            pl.BlockSpec((B, T, DEMB), lambda s, gref: (0, 0, 0)),
        ],
    )
    ofT, oaT, dye_sum = pl.pallas_call(
        _series_kernel,
        grid_spec=grid_spec,
        out_shape=[
            jax.ShapeDtypeStruct((N, B, SEQ), f32),
            jax.ShapeDtypeStruct((N, B, PRED), f32),
            jax.ShapeDtypeStruct((B, T, DEMB), f32),
        ],
    )(gi, jnp.transpose(x, (1, 0, 2)), dy, A, We, be.reshape(1, DEMB),
      Ws, bs.reshape(G, 1, PRED), Wt, bt.reshape(G, 1, PRED),
      W1, b1.reshape(G, 1, HID), W2, b2.reshape(G, 1, 1), gin, gout)
    of = jnp.transpose(ofT, (1, 0, 2))
    oa = jnp.transpose(oaT, (1, 0, 2))

    outg = pl.pallas_call(
        _gru_kernel,
        out_shape=jax.ShapeDtypeStruct((2 * B, SEQ), f32),
        scratch_shapes=[
            pltpu.VMEM((2 * B, SEQ, 3 * HID), f32),
            pltpu.VMEM((2 * B, SEQ, HID), f32),
        ],
    )(sf, tf, sas, tas, dye_sum, Wih, Whh, bih.reshape(1, 3 * HID),
      bhh.reshape(1, 3 * HID), Wa, ba.reshape(1, 1))
    oaf = outg[:B, :]
    oaa = outg[B:, :PRED]
    return (of, oaf, oa, oaa)
